# baseline (speedup 1.0000x reference)
import functools

import jax
import jax.numpy as jnp
from jax import lax
from jax.experimental import pallas as pl
from jax.experimental.pallas import tpu as pltpu
from jax.experimental.pallas import tpu_sc as plsc

N_TOKEN = 100000
D_EMBED = 128
N_TOK_BATCH = 16384
LANES = 16
NUM_CORES = 2
NUM_SUBCORES = 16
NUM_WORKERS = NUM_CORES * NUM_SUBCORES
B_PER_W = N_TOK_BATCH // NUM_WORKERS
SCALE = float(D_EMBED) ** 0.5

CHUNK = 256
NCHUNK = B_PER_W // CHUNK
NBUF = NCHUNK

_mesh = plsc.VectorSubcoreMesh(core_axis_name="c", subcore_axis_name="s")


@functools.partial(
    pl.kernel,
    mesh=_mesh,
    out_type=jax.ShapeDtypeStruct((N_TOK_BATCH, D_EMBED), jnp.float32),
    scratch_types=[
        pltpu.VMEM((B_PER_W,), jnp.int32),
        pltpu.VMEM((NBUF, CHUNK, D_EMBED), jnp.float32),
        pltpu.SemaphoreType.DMA((NBUF,)),
        pltpu.SemaphoreType.DMA((NBUF,)),
    ],
)
def _gather_scale(idx_hbm, table_hbm, out_hbm, idx_v, bufs, gsem, ssem):
    wid = lax.axis_index("s") * NUM_CORES + lax.axis_index("c")
    base = wid * B_PER_W
    pltpu.sync_copy(idx_hbm.at[pl.ds(base, B_PER_W)], idx_v)

    def gather(c):
        return pltpu.make_async_copy(
            table_hbm.at[idx_v.at[pl.ds(c * CHUNK, CHUNK)]],
            bufs.at[c % NBUF],
            gsem.at[c % NBUF],
        )

    def store(c):
        return pltpu.make_async_copy(
            bufs.at[c % NBUF],
            out_hbm.at[pl.ds(base + c * CHUNK, CHUNK)],
            ssem.at[c % NBUF],
        )

    for c in range(NCHUNK):
        gather(c).start()

    for c in range(NCHUNK):
        gather(c).wait()

        @plsc.parallel_loop(0, CHUNK, unroll=4)
        def _scale_rows(r):
            for j in range(D_EMBED // LANES):
                s = pl.ds(j * LANES, LANES)
                bufs[c % NBUF, r, s] = bufs[c % NBUF, r, s] * SCALE

        store(c).start()
    for c in range(NCHUNK):
        store(c).wait()


def kernel(x, labels, emb_weight, out_weight, out_bias, cluster_weight,
           cluster_bias):
    del labels, out_weight, out_bias, cluster_weight, cluster_bias
    return _gather_scale(x.astype(jnp.int32), emb_weight)


# speedup vs baseline: 1.0084x; 1.0084x over previous
import functools

import jax
import jax.numpy as jnp
from jax import lax
from jax.experimental import pallas as pl
from jax.experimental.pallas import tpu as pltpu
from jax.experimental.pallas import tpu_sc as plsc

N_TOKEN = 100000
D_EMBED = 128
N_TOK_BATCH = 16384
LANES = 16
NUM_CORES = 2
NUM_SUBCORES = 16
NUM_WORKERS = NUM_CORES * NUM_SUBCORES
B_PER_W = N_TOK_BATCH // NUM_WORKERS
SCALE = float(D_EMBED) ** 0.5

CHUNKS = (160, 160, 128, 64)
OFFS = (0, 160, 320, 448)
NCHUNK = len(CHUNKS)

_mesh = plsc.VectorSubcoreMesh(core_axis_name="c", subcore_axis_name="s")


@functools.partial(
    pl.kernel,
    mesh=_mesh,
    out_type=jax.ShapeDtypeStruct((N_TOK_BATCH, D_EMBED), jnp.float32),
    scratch_types=[
        pltpu.VMEM((B_PER_W,), jnp.int32),
        pltpu.VMEM((B_PER_W, D_EMBED), jnp.float32),
        pltpu.SemaphoreType.DMA((NCHUNK,)),
        pltpu.SemaphoreType.DMA((NCHUNK,)),
    ],
)
def _gather_scale(idx_hbm, table_hbm, out_hbm, idx_v, bufs, gsem, ssem):
    wid = lax.axis_index("s") * NUM_CORES + lax.axis_index("c")
    base = wid * B_PER_W
    pltpu.sync_copy(idx_hbm.at[pl.ds(base, B_PER_W)], idx_v)

    def gather(c):
        return pltpu.make_async_copy(
            table_hbm.at[idx_v.at[pl.ds(OFFS[c], CHUNKS[c])]],
            bufs.at[pl.ds(OFFS[c], CHUNKS[c])],
            gsem.at[c],
        )

    def store(c):
        return pltpu.make_async_copy(
            bufs.at[pl.ds(OFFS[c], CHUNKS[c])],
            out_hbm.at[pl.ds(base + OFFS[c], CHUNKS[c])],
            ssem.at[c],
        )

    for c in range(NCHUNK):
        gather(c).start()

    for c in range(NCHUNK):
        gather(c).wait()

        @plsc.parallel_loop(OFFS[c], OFFS[c] + CHUNKS[c], unroll=4)
        def _scale_rows(r):
            for j in range(D_EMBED // LANES):
                s = pl.ds(j * LANES, LANES)
                bufs[r, s] = bufs[r, s] * SCALE

        store(c).start()
    for c in range(NCHUNK):
        store(c).wait()


def kernel(x, labels, emb_weight, out_weight, out_bias, cluster_weight,
           cluster_bias):
    del labels, out_weight, out_bias, cluster_weight, cluster_bias
    return _gather_scale(x.astype(jnp.int32), emb_weight)


# speedup vs baseline: 1.0166x; 1.0082x over previous
import functools

import jax
import jax.numpy as jnp
from jax import lax
from jax.experimental import pallas as pl
from jax.experimental.pallas import tpu as pltpu
from jax.experimental.pallas import tpu_sc as plsc

N_TOKEN = 100000
D_EMBED = 128
N_TOK_BATCH = 16384
LANES = 16
NUM_CORES = 2
NUM_SUBCORES = 16
NUM_WORKERS = NUM_CORES * NUM_SUBCORES
B_PER_W = N_TOK_BATCH // NUM_WORKERS
SCALE = float(D_EMBED) ** 0.5

CHUNKS = (128, 128, 128, 128)
OFFS = (0, 128, 256, 384)
NCHUNK = len(CHUNKS)

_mesh = plsc.VectorSubcoreMesh(core_axis_name="c", subcore_axis_name="s")


@functools.partial(
    pl.kernel,
    mesh=_mesh,
    out_type=jax.ShapeDtypeStruct((N_TOK_BATCH, D_EMBED), jnp.float32),
    scratch_types=[
        pltpu.VMEM((B_PER_W,), jnp.int32),
        pltpu.VMEM((B_PER_W, D_EMBED), jnp.float32),
        pltpu.SemaphoreType.DMA((NCHUNK,)),
        pltpu.SemaphoreType.DMA((NCHUNK,)),
    ],
)
def _gather_scale(idx_hbm, table_hbm, out_hbm, idx_v, bufs, gsem, ssem):
    wid = lax.axis_index("s") * NUM_CORES + lax.axis_index("c")
    base = wid * B_PER_W
    pltpu.sync_copy(idx_hbm.at[pl.ds(base, B_PER_W)], idx_v)

    def gather(c):
        return pltpu.make_async_copy(
            table_hbm.at[idx_v.at[pl.ds(OFFS[c], CHUNKS[c])]],
            bufs.at[pl.ds(OFFS[c], CHUNKS[c])],
            gsem.at[c],
        )

    def store(c):
        return pltpu.make_async_copy(
            bufs.at[pl.ds(OFFS[c], CHUNKS[c])],
            out_hbm.at[pl.ds(base + OFFS[c], CHUNKS[c])],
            ssem.at[c],
        )

    for c in range(NCHUNK):
        gather(c).start()

    for c in range(NCHUNK):
        gather(c).wait()

        @plsc.parallel_loop(OFFS[c], OFFS[c] + CHUNKS[c], unroll=4)
        def _scale_rows(r):
            for j in range(D_EMBED // LANES):
                s = pl.ds(j * LANES, LANES)
                bufs[r, s] = bufs[r, s] * SCALE

        store(c).start()
    for c in range(NCHUNK):
        store(c).wait()


def kernel(x, labels, emb_weight, out_weight, out_bias, cluster_weight,
           cluster_bias):
    del labels, out_weight, out_bias, cluster_weight, cluster_bias
    return _gather_scale(x.astype(jnp.int32), emb_weight)
